# Initial kernel scaffold; baseline (speedup 1.0000x reference)
import functools

import jax
import jax.numpy as jnp
from jax import lax
from jax.experimental import pallas as pl
from jax.experimental.pallas import tpu as pltpu
from jax.experimental.pallas import tpu_sc as plsc

N = 10000
E = 320000
D = 128
NEG_SLOPE = 0.2

NC = 2
NS = 16
L = 16
NW = NC * NS
EW = E // NW
C = 80
NCHUNK = EW // C
RPT = N // NS


def _ab_body(h_ref, w_ref, o_ref):
    o_ref[...] = jnp.dot(h_ref[...], w_ref[...],
                         preferred_element_type=jnp.float32)


def _ab_call(h, w_pad):
    blk = 1000
    return pl.pallas_call(
        _ab_body,
        grid=(N // blk,),
        in_specs=[
            pl.BlockSpec((blk, D), lambda i: (i, 0)),
            pl.BlockSpec((D, 128), lambda i: (0, 0)),
        ],
        out_specs=pl.BlockSpec((blk, 128), lambda i: (i, 0)),
        out_shape=jax.ShapeDtypeStruct((N, 128), jnp.float32),
    )(h, w_pad)


def _sc_body(h_hbm, ei_hbm, a_hbm, b_hbm, part_hbm, den_hbm,
             src_v, dst_v, a_v, b_v, den_v, p_v, rows_v, acc, sem):
    cid = lax.axis_index("c")
    sid = lax.axis_index("s")
    wid = sid * NC + cid

    pltpu.sync_copy(ei_hbm.at[0, wid], src_v)
    pltpu.sync_copy(ei_hbm.at[1, wid], dst_v)
    pltpu.sync_copy(a_hbm, a_v)
    pltpu.sync_copy(b_hbm, b_v)

    def _zden(i, carry):
        den_v[pl.ds(i * L, L)] = jnp.zeros((L,), jnp.float32)
        return carry
    lax.fori_loop(0, N // L, _zden, 0)

    def _zrow(i, carry):
        for j in range(D // L):
            rows_v[i, pl.ds(j * L, L)] = jnp.zeros((L,), jnp.float32)
        return carry
    lax.fori_loop(0, C, _zrow, 0)
    base = sid * RPT
    for k in range(RPT // C):
        pltpu.sync_copy(rows_v, acc.at[pl.ds(base + k * C, C)])
    rem = RPT % C
    if rem:
        pltpu.sync_copy(rows_v.at[pl.ds(0, rem)],
                        acc.at[pl.ds(base + (RPT // C) * C, rem)])
    plsc.subcore_barrier()

    def _chunk(ci, carry):
        cp = pltpu.async_copy(h_hbm.at[src_v.at[ci]], rows_v, sem)
        for g in range(C // L):
            sv = src_v[ci, pl.ds(g * L, L)]
            dv = dst_v[ci, pl.ds(g * L, L)]
            e = plsc.load_gather(a_v, [sv]) + plsc.load_gather(b_v, [dv])
            e = jnp.where(e >= 0, e, NEG_SLOPE * e)
            p = jnp.exp(e)
            p_v[pl.ds(g * L, L)] = p
            plsc.addupdate_scatter(den_v, [dv], p)
        cp.wait()

        def _scale(i, carry2):
            pi = p_v[i]
            for j in range(D // L):
                rows_v[i, pl.ds(j * L, L)] = rows_v[i, pl.ds(j * L, L)] * pi
            return carry2
        lax.fori_loop(0, C, _scale, 0)

        pltpu.sync_copy(rows_v, acc.at[dst_v.at[ci]], add=True)
        return carry
    lax.fori_loop(0, NCHUNK, _chunk, 0)

    plsc.subcore_barrier()

    pltpu.sync_copy(acc.at[pl.ds(base, RPT)],
                    part_hbm.at[cid, pl.ds(base, RPT)])
    pltpu.sync_copy(den_v, den_hbm.at[wid])


def _sc_call(h, ei4, a, b):
    mesh = plsc.VectorSubcoreMesh(core_axis_name="c", subcore_axis_name="s",
                                  num_cores=NC, num_subcores=NS)
    fn = pl.kernel(
        _sc_body,
        out_type=(
            jax.ShapeDtypeStruct((NC, N, D), jnp.float32),
            jax.ShapeDtypeStruct((NW, N), jnp.float32),
        ),
        mesh=mesh,
        scratch_types=(
            pltpu.VMEM((NCHUNK, C), jnp.int32),
            pltpu.VMEM((NCHUNK, C), jnp.int32),
            pltpu.VMEM((N,), jnp.float32),
            pltpu.VMEM((N,), jnp.float32),
            pltpu.VMEM((N,), jnp.float32),
            pltpu.VMEM((C,), jnp.float32),
            pltpu.VMEM((C, D), jnp.float32),
            pltpu.VMEM_SHARED((N, D), jnp.float32),
            pltpu.SemaphoreType.DMA,
        ),
    )
    return fn(h, ei4, a, b)


def _fin_body(p_ref, d_ref, o_ref):
    s = p_ref[0] + p_ref[1]
    den = jnp.sum(d_ref[...], axis=0)
    o_ref[...] = s / jnp.maximum(den, 1e-9)[:, None]


def _fin_call(partials, denoms):
    blk = 500
    return pl.pallas_call(
        _fin_body,
        grid=(N // blk,),
        in_specs=[
            pl.BlockSpec((NC, blk, D), lambda i: (0, i, 0)),
            pl.BlockSpec((NW, blk), lambda i: (0, i)),
        ],
        out_specs=pl.BlockSpec((blk, D), lambda i: (i, 0)),
        out_shape=jax.ShapeDtypeStruct((N, D), jnp.float32),
    )(partials, denoms)


@jax.jit
def kernel(h, edge_index, W_att):
    w_row = W_att[0]
    w_pad = jnp.zeros((D, 128), jnp.float32)
    w_pad = w_pad.at[:, 0].set(w_row[:D]).at[:, 1].set(w_row[D:])
    ab = _ab_call(h, w_pad)
    a = ab[:, 0]
    b = ab[:, 1]
    ei4 = edge_index.reshape(2, NW, NCHUNK, C)
    partials, denoms = _sc_call(h, ei4, a, b)
    return _fin_call(partials, denoms)


# speedup vs baseline: 20.6141x; 20.6141x over previous
import functools

import jax
import jax.numpy as jnp
from jax import lax
from jax.experimental import pallas as pl
from jax.experimental.pallas import tpu as pltpu
from jax.experimental.pallas import tpu_sc as plsc

N = 10000
E = 320000
D = 128
NEG_SLOPE = 0.2

NC = 2
NS = 16
L = 16
NW = NC * NS
EW = E // NW
C = 80
NCHUNK = EW // C
RPT8 = 624
DEN_R = 80


def _ab_body(h_ref, w_ref, o_ref):
    o_ref[...] = jnp.dot(h_ref[...], w_ref[...],
                         preferred_element_type=jnp.float32)


def _ab_call(h, w_pad):
    blk = 1000
    return pl.pallas_call(
        _ab_body,
        grid=(N // blk,),
        in_specs=[
            pl.BlockSpec((blk, D), lambda i: (i, 0)),
            pl.BlockSpec((D, 128), lambda i: (0, 0)),
        ],
        out_specs=pl.BlockSpec((blk, 128), lambda i: (i, 0)),
        out_shape=jax.ShapeDtypeStruct((N, 128), jnp.float32),
    )(h, w_pad)


def _sc_body(h_hbm, src_hbm, dst_hbm, a_hbm, b_hbm, part_hbm, den_hbm,
             src_v, dst_v, a_v, b_v, den_v, p_v, rows_v, acc, sem, isem):
    cid = lax.axis_index("c")
    sid = lax.axis_index("s")
    wid = sid * NC + cid
    ebase = wid * EW

    pltpu.sync_copy(a_hbm, a_v.at[pl.ds(0, N)])
    pltpu.sync_copy(b_hbm, b_v.at[pl.ds(0, N)])

    def _zden(i, carry):
        for j in range(D // L):
            den_v[i, pl.ds(j * L, L)] = jnp.zeros((L,), jnp.float32)
        return carry
    lax.fori_loop(0, DEN_R, _zden, 0)

    def _zrow(i, carry):
        for j in range(D // L):
            rows_v[i, pl.ds(j * L, L)] = jnp.zeros((L,), jnp.float32)
        return carry
    lax.fori_loop(0, C, _zrow, 0)
    base = sid * RPT8
    for k in range(RPT8 // C):
        pltpu.sync_copy(rows_v, acc.at[pl.ds(base + k * C, C)])
    rem = RPT8 % C
    if rem:
        pltpu.sync_copy(rows_v.at[pl.ds(0, rem)],
                        acc.at[pl.ds(base + (RPT8 // C) * C, rem)])

    @pl.when(sid == NS - 1)
    def _tail_zero():
        pltpu.sync_copy(rows_v.at[pl.ds(0, N - NS * RPT8)],
                        acc.at[pl.ds(NS * RPT8, N - NS * RPT8)])
    plsc.subcore_barrier()

    pltpu.async_copy(src_hbm.at[pl.ds(ebase, C)], src_v.at[0], isem).wait()
    pltpu.async_copy(dst_hbm.at[pl.ds(ebase, C)], dst_v.at[0], isem).wait()

    def _chunk(ci, carry):
        slot = lax.rem(ci, 2)
        nslot = 1 - slot
        nci = jnp.minimum(ci + 1, NCHUNK - 1)
        cpn_s = pltpu.async_copy(src_hbm.at[pl.ds(ebase + nci * C, C)],
                                 src_v.at[nslot], isem)
        cpn_d = pltpu.async_copy(dst_hbm.at[pl.ds(ebase + nci * C, C)],
                                 dst_v.at[nslot], isem)
        cp = pltpu.async_copy(h_hbm.at[src_v.at[slot]], rows_v, sem)
        for g in range(C // L):
            sv = src_v[slot, pl.ds(g * L, L)]
            dv = dst_v[slot, pl.ds(g * L, L)]
            e = plsc.load_gather(a_v, [sv]) + plsc.load_gather(b_v, [dv])
            e = jnp.where(e >= 0, e, NEG_SLOPE * e)
            p = jnp.exp(e)
            p_v[pl.ds(g * L, L)] = p
            plsc.addupdate_scatter(
                den_v, [jax.lax.shift_right_logical(dv, 7),
                        jnp.bitwise_and(dv, 127)], p)
        cp.wait()

        def _scale(i, carry2):
            pi = p_v[pl.ds(i, L)][0]
            for j in range(D // L):
                rows_v[i, pl.ds(j * L, L)] = rows_v[i, pl.ds(j * L, L)] * pi
            return carry2
        lax.fori_loop(0, C, _scale, 0)

        pltpu.sync_copy(rows_v, acc.at[dst_v.at[slot]], add=True)
        cpn_s.wait()
        cpn_d.wait()
        return carry
    lax.fori_loop(0, NCHUNK, _chunk, 0)

    plsc.subcore_barrier()

    pltpu.sync_copy(acc.at[pl.ds(base, RPT8)],
                    part_hbm.at[cid, pl.ds(base, RPT8)])

    @pl.when(sid == NS - 1)
    def _tail_out():
        pltpu.sync_copy(acc.at[pl.ds(NS * RPT8, N - NS * RPT8)],
                        part_hbm.at[cid, pl.ds(NS * RPT8, N - NS * RPT8)])

    pltpu.sync_copy(den_v, den_hbm.at[wid])


def _sc_call(h, src, dst, a, b):
    mesh = plsc.VectorSubcoreMesh(core_axis_name="c", subcore_axis_name="s",
                                  num_cores=NC, num_subcores=NS)
    fn = pl.kernel(
        _sc_body,
        out_type=(
            jax.ShapeDtypeStruct((NC, N, D), jnp.float32),
            jax.ShapeDtypeStruct((NW, DEN_R, 128), jnp.float32),
        ),
        mesh=mesh,
        compiler_params=pltpu.CompilerParams(needs_layout_passes=False,
                                             use_tc_tiling_on_sc=False),
        scratch_types=(
            pltpu.VMEM((2, C), jnp.int32),
            pltpu.VMEM((2, C), jnp.int32),
            pltpu.VMEM((DEN_R * 128,), jnp.float32),
            pltpu.VMEM((DEN_R * 128,), jnp.float32),
            pltpu.VMEM((DEN_R, 128), jnp.float32),
            pltpu.VMEM((C + L,), jnp.float32),
            pltpu.VMEM((C, D), jnp.float32),
            pltpu.VMEM_SHARED((N, D), jnp.float32),
            pltpu.SemaphoreType.DMA,
            pltpu.SemaphoreType.DMA,
        ),
    )
    return fn(h, src, dst, a, b)


def _fin_body(p0_ref, p1_ref, d_ref, o_ref):
    s = p0_ref[0] + p1_ref[0]
    den = jnp.sum(d_ref[...], axis=1)
    o_ref[...] = s / jnp.maximum(den, 1e-9)[:, None]


def _fin_call(partials, denoms_t):
    blk = 400
    return pl.pallas_call(
        _fin_body,
        grid=(N // blk,),
        in_specs=[
            pl.BlockSpec((1, blk, D), lambda i: (0, i, 0)),
            pl.BlockSpec((1, blk, D), lambda i: (1, i, 0)),
            pl.BlockSpec((blk, NW), lambda i: (i, 0)),
        ],
        out_specs=pl.BlockSpec((blk, D), lambda i: (i, 0)),
        out_shape=jax.ShapeDtypeStruct((N, D), jnp.float32),
    )(partials, partials, denoms_t)


@jax.jit
def kernel(h, edge_index, W_att):
    w_row = W_att[0]
    w_pad = jnp.zeros((D, 128), jnp.float32)
    w_pad = w_pad.at[:, 0].set(w_row[:D]).at[:, 1].set(w_row[D:])
    ab = _ab_call(h, w_pad)
    a = ab[:, 0]
    b = ab[:, 1]
    partials, denoms = _sc_call(h, edge_index[0], edge_index[1], a, b)
    den_t = denoms.reshape(NW, DEN_R * 128)[:, :N].T
    return _fin_call(partials, den_t)
